# Initial kernel scaffold; baseline (speedup 1.0000x reference)
import jax
import jax.numpy as jnp
from jax.experimental import pallas as pl


def kernel(x, W, b):
    raise NotImplementedError("write your pallas kernel here")


# speedup vs baseline: 1.4179x; 1.4179x over previous
import jax
import jax.numpy as jnp
from jax.experimental import pallas as pl
from jax.experimental.pallas import tpu as pltpu

K = 8
ALPHA = 0.01
CAPACITY_FACTOR = 1.25
B = 1024


def _make_body(n_experts, n_tokens, cap, grid):
    def _body(x_ref, w_ref, b_ref, probs_ref, idx_ref, mask_ref, loss_ref,
              acc_p, acc_a, tri_ref):
        step = pl.program_id(0)
        N = n_experts

        @pl.when(step == 0)
        def _init():
            acc_p[...] = jnp.zeros_like(acc_p)
            acc_a[...] = jnp.zeros_like(acc_a)
            row = jax.lax.broadcasted_iota(jnp.int32, (B, B), 0)
            col = jax.lax.broadcasted_iota(jnp.int32, (B, B), 1)
            tri_ref[...] = (row >= col).astype(jnp.bfloat16)

        logits = jnp.dot(x_ref[...], w_ref[...],
                         preferred_element_type=jnp.float32)
        logits = logits + b_ref[...]
        m = jnp.max(logits, axis=1, keepdims=True)
        e = jnp.exp(logits - m)
        gate = e / jnp.sum(e, axis=1, keepdims=True)

        lane = jax.lax.broadcasted_iota(jnp.int32, (B, N), 1)
        p = gate
        active = jnp.zeros((B, N), jnp.float32)
        for k in range(K):
            mk = jnp.max(p, axis=1, keepdims=True)
            ik = jnp.min(jnp.where(p == mk, lane, N), axis=1, keepdims=True)
            sel = lane == ik
            probs_ref[:, k:k + 1] = mk
            idx_ref[:, k:k + 1] = ik
            active = jnp.where(sel, 1.0, active)
            p = jnp.where(sel, -1.0, p)

        acc_p[...] += jnp.sum(gate, axis=0, keepdims=True)
        carry = acc_a[...]
        acc_a[...] = carry + jnp.sum(active, axis=0, keepdims=True)

        csum = jnp.dot(tri_ref[...], active.astype(jnp.bfloat16),
                       preferred_element_type=jnp.float32)
        rank_t = jnp.transpose(csum + carry)
        act_t = jnp.transpose(active)
        mask_ref[...] = (act_t > 0.5) & (rank_t <= cap)

        @pl.when(step == grid - 1)
        def _fin():
            s = jnp.sum(acc_p[...] * acc_a[...])
            loss_ref[0, 0] = ALPHA * N * s / (n_tokens * n_tokens)

    return _body


def kernel(x, W, b):
    T, D = x.shape
    N = W.shape[1]
    cap = int(CAPACITY_FACTOR * T / N)
    grid = T // B

    probs, idx, mask, loss = pl.pallas_call(
        _make_body(N, T, cap, grid),
        grid=(grid,),
        in_specs=[
            pl.BlockSpec((B, D), lambda i: (i, 0)),
            pl.BlockSpec((D, N), lambda i: (0, 0)),
            pl.BlockSpec((1, N), lambda i: (0, 0)),
        ],
        out_specs=[
            pl.BlockSpec((B, K), lambda i: (i, 0)),
            pl.BlockSpec((B, K), lambda i: (i, 0)),
            pl.BlockSpec((N, B), lambda i: (0, i)),
            pl.BlockSpec((1, 1), lambda i: (0, 0),
                         memory_space=pltpu.SMEM),
        ],
        out_shape=[
            jax.ShapeDtypeStruct((T, K), jnp.float32),
            jax.ShapeDtypeStruct((T, K), jnp.int32),
            jax.ShapeDtypeStruct((N, T), jnp.bool_),
            jax.ShapeDtypeStruct((1, 1), jnp.float32),
        ],
        scratch_shapes=[
            pltpu.VMEM((1, N), jnp.float32),
            pltpu.VMEM((1, N), jnp.float32),
            pltpu.VMEM((B, B), jnp.bfloat16),
        ],
        compiler_params=pltpu.CompilerParams(
            dimension_semantics=("arbitrary",)),
    )(x, W, b.reshape(1, N))
    return (loss[0, 0], probs, idx, mask)
